# Initial kernel scaffold; baseline (speedup 1.0000x reference)
import jax
import jax.numpy as jnp
from jax.experimental import pallas as pl


def kernel(x, embedding_table):
    raise NotImplementedError("write your pallas kernel here")


# speedup vs baseline: 1.9020x; 1.9020x over previous
import functools

import jax
import jax.numpy as jnp
import numpy as np
from jax import lax
from jax.experimental import pallas as pl
from jax.experimental.pallas import tpu as pltpu
from jax.experimental.pallas import tpu_sc as plsc

VOCAB = 100000
SENT = 200
DIM = 64
BATCH = 1024

NC = 2
NS = 16
NW = NC * NS
LANES = 16

N_ROWS = BATCH * SENT
ROWS_PER_W = N_ROWS // NW
CHUNK = 128
N_CHUNKS = ROWS_PER_W // CHUNK
D_SLICES = DIM // LANES
SCALE = 8.0


def _positional_encoding(length, depth):
    depth = depth / 2
    positions = np.arange(length)[:, np.newaxis]
    depths = np.arange(depth)[np.newaxis, :] / depth
    angle_rates = 1 / 10000 ** depths
    angle_rads = positions * angle_rates
    return np.concatenate([np.sin(angle_rads), np.cos(angle_rads)], axis=-1).astype(np.float32)

_POS_NP = _positional_encoding(SENT, DIM)


def _sc_body(table_hbm, x_hbm, pos_hbm, emb_hbm, mask_hbm,
             pos_v, idx_v, rows_v, mask_v, sem):
    wid = lax.axis_index("s") * NC + lax.axis_index("c")
    base_w = wid * ROWS_PER_W

    pltpu.sync_copy(pos_hbm, pos_v)

    def chunk_body(c, _):
        base = base_w + c * CHUNK
        pltpu.sync_copy(x_hbm.at[pl.ds(base, CHUNK)], idx_v)
        pltpu.async_copy(table_hbm.at[idx_v], rows_v, sem).wait()

        def row_body(r, _):
            p = lax.rem(base + r, SENT)
            for d in range(D_SLICES):
                sl = pl.ds(d * LANES, LANES)
                rows_v[r, sl] = rows_v[r, sl] * SCALE + pos_v[p, sl]
            return _

        lax.fori_loop(0, CHUNK, row_body, 0)

        for m in range(CHUNK // LANES):
            sl = pl.ds(m * LANES, LANES)
            mask_v[sl] = jnp.where(idx_v[sl] != 0, 1, 0).astype(jnp.int32)

        pltpu.sync_copy(rows_v, emb_hbm.at[pl.ds(base, CHUNK)])
        pltpu.sync_copy(mask_v, mask_hbm.at[pl.ds(base, CHUNK)])
        return _

    lax.fori_loop(0, N_CHUNKS, chunk_body, 0)


@jax.jit
def _decode_embedding(x_flat, table, pos):
    mesh = plsc.VectorSubcoreMesh(
        core_axis_name="c", subcore_axis_name="s",
        num_cores=NC, num_subcores=NS)
    run = pl.kernel(
        _sc_body,
        out_type=(
            jax.ShapeDtypeStruct((N_ROWS, DIM), jnp.float32),
            jax.ShapeDtypeStruct((N_ROWS,), jnp.int32),
        ),
        mesh=mesh,
        scratch_types=[
            pltpu.VMEM((SENT, DIM), jnp.float32),
            pltpu.VMEM((CHUNK,), jnp.int32),
            pltpu.VMEM((CHUNK, DIM), jnp.float32),
            pltpu.VMEM((CHUNK,), jnp.int32),
            pltpu.SemaphoreType.DMA,
        ],
        compiler_params=pltpu.CompilerParams(use_tc_tiling_on_sc=False),
    )
    return run(table, x_flat, pos)


def kernel(x, embedding_table):
    pos = jnp.asarray(_POS_NP)
    emb_flat, mask_flat = _decode_embedding(
        x.reshape(-1), embedding_table, pos)
    return (emb_flat.reshape(BATCH, SENT, DIM),
            mask_flat.reshape(BATCH, SENT).astype(jnp.int32))


# speedup vs baseline: 3.1731x; 1.6683x over previous
import jax
import jax.numpy as jnp
import numpy as np
from jax import lax
from jax.experimental import pallas as pl
from jax.experimental.pallas import tpu as pltpu
from jax.experimental.pallas import tpu_sc as plsc

VOCAB = 100000
SENT = 200
DIM = 64
BATCH = 1024

NC = 2
NS = 16
NW = NC * NS
LANES = 16

N_ROWS = BATCH * SENT
ROWS_PER_W = N_ROWS // NW
CHUNK = 2 * SENT
N_CHUNKS = ROWS_PER_W // CHUNK
SUB = 80
N_SUB = CHUNK // SUB
D_SLICES = DIM // LANES
NBUF = 3
SCALE = 8.0


def _positional_encoding(length, depth):
    depth = depth / 2
    positions = np.arange(length)[:, np.newaxis]
    depths = np.arange(depth)[np.newaxis, :] / depth
    angle_rates = 1 / 10000 ** depths
    angle_rads = positions * angle_rates
    return np.concatenate([np.sin(angle_rads), np.cos(angle_rads)], axis=-1).astype(np.float32)

_POS_NP = _positional_encoding(SENT, DIM)


def _sc_body(table_hbm, x_hbm, pos_hbm, emb_hbm, mask_hbm,
             pos_v, idx_v, rows_v, mask_v, gsems, ssems):
    wid = lax.axis_index("s") * NC + lax.axis_index("c")
    base_w = wid * ROWS_PER_W

    pltpu.sync_copy(pos_hbm, pos_v)

    def gather_descs(b):
        return [pltpu.make_async_copy(
                    table_hbm.at[idx_v[b].at[pl.ds(k * SUB, SUB)]],
                    rows_v[b].at[pl.ds(k * SUB, SUB)],
                    gsems[b]) for k in range(N_SUB)]

    def store_descs(c, b):
        base = base_w + c * CHUNK
        return [pltpu.make_async_copy(rows_v[b], emb_hbm.at[pl.ds(base, CHUNK)], ssems[b]),
                pltpu.make_async_copy(mask_v[b], mask_hbm.at[pl.ds(base, CHUNK)], ssems[b])]

    def fire_fetch(c, b):
        base = base_w + c * CHUNK
        pltpu.sync_copy(x_hbm.at[pl.ds(base, CHUNK)], idx_v[b])
        for d in gather_descs(b):
            d.start()

    def compute(b):
        def mask_body(m, _):
            sl = pl.ds(m * LANES, LANES)
            mask_v[b][sl] = jnp.where(idx_v[b][sl] != 0, 1, 0).astype(jnp.int32)
            return _
        lax.fori_loop(0, CHUNK // LANES, mask_body, 0)

        def row_body(r, _):
            for d in range(D_SLICES):
                sl = pl.ds(d * LANES, LANES)
                pv = pos_v[r, sl]
                rows_v[b][r, sl] = rows_v[b][r, sl] * SCALE + pv
                rows_v[b][r + SENT, sl] = rows_v[b][r + SENT, sl] * SCALE + pv
            return _
        lax.fori_loop(0, SENT, row_body, 0)

    for c in range(N_CHUNKS):
        b = c % NBUF
        if c >= NBUF:
            for d in store_descs(c - NBUF, b):
                d.wait()
        fire_fetch(c, b)
        if c >= 1:
            bp = (c - 1) % NBUF
            for d in gather_descs(bp):
                d.wait()
            compute(bp)
            for d in store_descs(c - 1, bp):
                d.start()

    bl = (N_CHUNKS - 1) % NBUF
    for d in gather_descs(bl):
        d.wait()
    compute(bl)
    for d in store_descs(N_CHUNKS - 1, bl):
        d.start()
    for c in range(N_CHUNKS - NBUF, N_CHUNKS):
        for d in store_descs(c, c % NBUF):
            d.wait()


@jax.jit
def _decode_embedding(x_flat, table, pos):
    mesh = plsc.VectorSubcoreMesh(
        core_axis_name="c", subcore_axis_name="s",
        num_cores=NC, num_subcores=NS)
    run = pl.kernel(
        _sc_body,
        out_type=(
            jax.ShapeDtypeStruct((N_ROWS, DIM), jnp.float32),
            jax.ShapeDtypeStruct((N_ROWS,), jnp.int32),
        ),
        mesh=mesh,
        scratch_types=[
            pltpu.VMEM((SENT, DIM), jnp.float32),
            [pltpu.VMEM((CHUNK,), jnp.int32) for _ in range(NBUF)],
            [pltpu.VMEM((CHUNK, DIM), jnp.float32) for _ in range(NBUF)],
            [pltpu.VMEM((CHUNK,), jnp.int32) for _ in range(NBUF)],
            [pltpu.SemaphoreType.DMA for _ in range(NBUF)],
            [pltpu.SemaphoreType.DMA for _ in range(NBUF)],
        ],
        compiler_params=pltpu.CompilerParams(use_tc_tiling_on_sc=False),
    )
    return run(table, x_flat, pos)


def kernel(x, embedding_table):
    pos = jnp.asarray(_POS_NP)
    emb_flat, mask_flat = _decode_embedding(
        x.reshape(-1), embedding_table, pos)
    return (emb_flat.reshape(BATCH, SENT, DIM),
            mask_flat.reshape(BATCH, SENT).astype(jnp.int32))
